# baseline (speedup 1.0000x reference)
import jax
import jax.numpy as jnp
from jax.experimental import pallas as pl

ALPHA = 8
FRAMES = 64


def _pack_body(x_ref, slow_ref, fast_ref):
    fast_ref[...] = x_ref[...]
    for j in range(8):
        slow_ref[0, j] = x_ref[0, 9 * j]


def kernel(x):
    C, T, H, W = x.shape
    G = T // ALPHA
    NG = T // FRAMES
    slow, fast = pl.pallas_call(
        _pack_body,
        grid=(C, 2),
        in_specs=[
            pl.BlockSpec((1, FRAMES, H // 2, W), lambda c, hh: (c, 0, hh, 0)),
        ],
        out_specs=[
            pl.BlockSpec((1, ALPHA, H // 2, W), lambda c, hh: (c, 0, hh, 0)),
            pl.BlockSpec((1, FRAMES, H // 2, W), lambda c, hh: (c, 0, hh, 0)),
        ],
        out_shape=[
            jax.ShapeDtypeStruct((C, G, H, W), x.dtype),
            jax.ShapeDtypeStruct((C, T, H, W), x.dtype),
        ],
    )(x)
    return (slow, fast)
